# baseline (speedup 1.0000x reference)
import functools

import jax
import jax.numpy as jnp
from jax import lax
from jax.experimental import pallas as pl
from jax.experimental.pallas import tpu as pltpu
from jax.experimental.pallas import tpu_sc as plsc

N_NODES = 100000
N_EDGES = 1600000
H = 32
HH = 16
N_P = 100096
N8 = N_P // 8

NC = 2
NS = 16
LANE = 128

ROWS_PER_TILE = 786
K = 3
NBUF = 4
CHUNKS = ROWS_PER_TILE // K
R_PAD = ROWS_PER_TILE * NS
R_ALLOC = R_PAD + K
E_ALLOC = R_ALLOC * LANE
PAD = E_ALLOC - N_EDGES

ACC_ROWS = N_P + NS
STRIPE = N_P // NS


def _agg_body(h_hbm, src_hbm, dst_hbm, out_hbm, sidx, didx, rows, acc, gsem, ssem, isem):
    c = lax.axis_index("c")
    s = lax.axis_index("s")

    h_tab = h_hbm.at[c]
    out_tab = out_hbm.at[c]

    pltpu.sync_copy(h_tab.at[pl.ds(s * STRIPE, STRIPE)],
                    acc.at[pl.ds(s * STRIPE, STRIPE)])

    plsc.subcore_barrier()

    base0 = s * ROWS_PER_TILE

    def fire_idx(j, p):
        base = base0 + j * K
        pltpu.async_copy(src_hbm.at[pl.ds(base, K)], sidx.at[p], isem)
        pltpu.async_copy(dst_hbm.at[pl.ds(base, K)], didx.at[p], isem)

    def wait_idx(p):
        for buf in (sidx, didx):
            pltpu.make_async_copy(src_hbm.at[pl.ds(0, K)], buf.at[p], isem).wait()

    def fire_gathers(p):
        for k in range(K):
            pltpu.async_copy(h_tab.at[sidx.at[p, k]], rows.at[p, k], gsem)

    def wait_gathers(p):
        for k in range(K):
            pltpu.make_async_copy(h_tab.at[pl.ds(0, LANE)],
                                  rows.at[p, k], gsem).wait()

    def fire_scatters(p):
        for k in range(K):
            pltpu.async_copy(rows.at[p, k], acc.at[didx.at[p, k]], ssem, add=True)

    def drain_scatters(p):
        for k in range(K):
            pltpu.make_async_copy(h_tab.at[pl.ds(0, LANE)],
                                  rows.at[p, k], ssem).wait()

    fire_idx(0, 0)
    wait_idx(0)
    fire_gathers(0)
    fire_idx(1, 1)

    def step(j, carry):
        p = lax.rem(j, NBUF)
        pn = lax.rem(j + 1, NBUF)
        pi = lax.rem(j + 2, NBUF)

        @pl.when(j >= 2)
        def _():
            drain_scatters(pi)
        wait_gathers(p)
        fire_scatters(p)
        wait_idx(pn)
        fire_gathers(pn)

        @pl.when(j + 2 < CHUNKS)
        def _():
            fire_idx(j + 2, pi)
        return carry

    lax.fori_loop(0, CHUNKS - 1, step, 0)

    jl = CHUNKS - 1
    pl_ = lax.rem(jnp.int32(jl), NBUF)
    drain_scatters(lax.rem(jnp.int32(jl - 2), NBUF))
    wait_gathers(pl_)
    fire_scatters(pl_)
    drain_scatters(lax.rem(jnp.int32(jl - 1), NBUF))
    drain_scatters(pl_)

    plsc.subcore_barrier()

    pltpu.sync_copy(acc.at[pl.ds(s * STRIPE, STRIPE)],
                    out_tab.at[pl.ds(s * STRIPE, STRIPE)])


@functools.lru_cache(maxsize=1)
def _make_agg():
    mesh = plsc.VectorSubcoreMesh(core_axis_name="c", subcore_axis_name="s",
                                  num_cores=NC, num_subcores=NS)
    return pl.kernel(
        _agg_body,
        out_type=jax.ShapeDtypeStruct((NC, N_P, HH), jnp.float32),
        mesh=mesh,
        compiler_params=pltpu.CompilerParams(use_tc_tiling_on_sc=False),
        scratch_types=[
            pltpu.VMEM((NBUF, K, LANE), jnp.int32),
            pltpu.VMEM((NBUF, K, LANE), jnp.int32),
            pltpu.VMEM((NBUF, K, LANE, HH), jnp.float32),
            pltpu.VMEM_SHARED((ACC_ROWS, HH), jnp.float32),
            pltpu.SemaphoreType.DMA,
            pltpu.SemaphoreType.DMA,
            pltpu.SemaphoreType.DMA,
        ],
    )


BN = 6256
GRID = N_P // BN


def _mlp_math(z, w1, b1, w2, b2):
    a = jnp.maximum(jnp.dot(z, w1, preferred_element_type=jnp.float32) + b1, 0.0)
    return jnp.maximum(jnp.dot(a, w2, preferred_element_type=jnp.float32) + b2, 0.0)


def _mlp_body(z_ref, w1_ref, b1_ref, w2_ref, b2_ref, out_ref):
    z = jnp.concatenate([z_ref[0], z_ref[1]], axis=-1)
    h = _mlp_math(z, w1_ref[...], b1_ref[...], w2_ref[...], b2_ref[...])
    out_ref[0] = h[:, :HH]
    out_ref[1] = h[:, HH:]


def _final_body(z_ref, w1_ref, b1_ref, w2_ref, b2_ref, wf_ref, bf_ref, out_ref):
    z = jnp.concatenate([z_ref[0], z_ref[1]], axis=-1)
    h = _mlp_math(z, w1_ref[...], b1_ref[...], w2_ref[...], b2_ref[...])
    out_ref[...] = jnp.dot(h, wf_ref[...], preferred_element_type=jnp.float32) \
        + bf_ref[...]


_Z_SPEC = pl.BlockSpec((NC, BN, HH), lambda i: (0, i, 0))
_W_SPEC = pl.BlockSpec((H, H), lambda i: (0, 0))
_B_SPEC = pl.BlockSpec((1, H), lambda i: (0, 0))
_P_SPEC = pl.BlockSpec((NC, BN, HH), lambda i: (0, i, 0))
_P_SHAPE = jax.ShapeDtypeStruct((NC, N_P, HH), jnp.float32)

_mlp = pl.pallas_call(
    _mlp_body,
    grid=(GRID,),
    in_specs=[_Z_SPEC, _W_SPEC, _B_SPEC, _W_SPEC, _B_SPEC],
    out_specs=_P_SPEC,
    out_shape=_P_SHAPE,
)

_final = pl.pallas_call(
    _final_body,
    grid=(GRID,),
    in_specs=[_Z_SPEC, _W_SPEC, _B_SPEC, _W_SPEC, _B_SPEC,
              pl.BlockSpec((H, 1), lambda i: (0, 0)),
              pl.BlockSpec((1, 1), lambda i: (0, 0))],
    out_specs=pl.BlockSpec((BN, 1), lambda i: (i, 0)),
    out_shape=jax.ShapeDtypeStruct((N_NODES, 1), jnp.float32),
)


def kernel(x, edge_index, W1_0, b1_0, W2_0, b2_0, W1_1, b1_1, W2_1, b2_1,
           W1_2, b1_2, W2_2, b2_2, Wf, bf):
    lane_ids = jnp.arange(PAD, dtype=jnp.int32) % NS
    src = jnp.concatenate([edge_index[0], lane_ids]).reshape(R_ALLOC, LANE)
    dst = jnp.concatenate([edge_index[1], lane_ids + N_P]).reshape(R_ALLOC, LANE)

    xp = jnp.pad(x, ((0, N_P - N_NODES), (0, 0)))
    hs = jnp.stack([xp[:, :HH], xp[:, HH:]])
    b1s = (b1_0.reshape(1, H), b1_1.reshape(1, H), b1_2.reshape(1, H))
    b2s = (b2_0.reshape(1, H), b2_1.reshape(1, H), b2_2.reshape(1, H))
    W1s = (W1_0, W1_1, W1_2)
    W2s = (W2_0, W2_1, W2_2)
    _agg = _make_agg()

    for i in range(2):
        z = _agg(hs, src, dst)
        hs = _mlp(z, W1s[i], b1s[i], W2s[i], b2s[i])
    z = _agg(hs, src, dst)
    return _final(z, W1s[2], b1s[2], W2s[2], b2s[2], Wf, bf.reshape(1, 1))


# speedup vs baseline: 1.0906x; 1.0906x over previous
import functools

import jax
import jax.numpy as jnp
from jax import lax
from jax.experimental import pallas as pl
from jax.experimental.pallas import tpu as pltpu
from jax.experimental.pallas import tpu_sc as plsc

N_NODES = 100000
N_EDGES = 1600000
H = 32
HH = 16
N_P = 100096
N8 = N_P // 8

NC = 2
NS = 16
LANE = 128

ROWS_PER_TILE = 784
K = 4
NBUF = 3
CHUNKS = ROWS_PER_TILE // K
R_PAD = ROWS_PER_TILE * NS
R_ALLOC = R_PAD + K
E_ALLOC = R_ALLOC * LANE
PAD = E_ALLOC - N_EDGES

ACC_ROWS = N_P + NS
STRIPE = N_P // NS


def _agg_body(h_hbm, src_hbm, dst_hbm, out_hbm, sidx, didx, rows, acc, gsem, ssem, isem):
    c = lax.axis_index("c")
    s = lax.axis_index("s")

    h_tab = h_hbm.at[c]
    out_tab = out_hbm.at[c]

    pltpu.sync_copy(h_tab.at[pl.ds(s * STRIPE, STRIPE)],
                    acc.at[pl.ds(s * STRIPE, STRIPE)])

    plsc.subcore_barrier()

    base0 = s * ROWS_PER_TILE

    def fire_idx(j, p):
        base = base0 + j * K
        pltpu.async_copy(src_hbm.at[pl.ds(base, K)], sidx.at[p], isem)
        pltpu.async_copy(dst_hbm.at[pl.ds(base, K)], didx.at[p], isem)

    def wait_idx(p):
        for buf in (sidx, didx):
            pltpu.make_async_copy(src_hbm.at[pl.ds(0, K)], buf.at[p], isem).wait()

    def fire_gathers(p):
        for k in range(K):
            pltpu.async_copy(h_tab.at[sidx.at[p, k]], rows.at[p, k], gsem)

    def wait_gathers(p):
        for k in range(K):
            pltpu.make_async_copy(h_tab.at[pl.ds(0, LANE)],
                                  rows.at[p, k], gsem).wait()

    def fire_scatters(p):
        for k in range(K):
            pltpu.async_copy(rows.at[p, k], acc.at[didx.at[p, k]], ssem, add=True)

    def drain_scatters(p):
        for k in range(K):
            pltpu.make_async_copy(h_tab.at[pl.ds(0, LANE)],
                                  rows.at[p, k], ssem).wait()

    fire_idx(0, 0)
    wait_idx(0)
    fire_gathers(0)
    fire_idx(1, 1)

    def step(j, carry):
        p = lax.rem(j, NBUF)
        pn = lax.rem(j + 1, NBUF)
        pi = lax.rem(j + 2, NBUF)
        pd = lax.rem(j - 2 + NBUF, NBUF)

        @pl.when(j >= 2)
        def _():
            drain_scatters(pd)
        wait_gathers(p)
        fire_scatters(p)
        wait_idx(pn)
        fire_gathers(pn)

        @pl.when(j + 2 < CHUNKS)
        def _():
            fire_idx(j + 2, pi)
        return carry

    lax.fori_loop(0, CHUNKS - 1, step, 0)

    jl = CHUNKS - 1
    pl_ = lax.rem(jnp.int32(jl), NBUF)
    drain_scatters(lax.rem(jnp.int32(jl - 2 + NBUF), NBUF))
    wait_gathers(pl_)
    fire_scatters(pl_)
    drain_scatters(lax.rem(jnp.int32(jl - 1), NBUF))
    drain_scatters(pl_)

    plsc.subcore_barrier()

    pltpu.sync_copy(acc.at[pl.ds(s * STRIPE, STRIPE)],
                    out_tab.at[pl.ds(s * STRIPE, STRIPE)])


@functools.lru_cache(maxsize=1)
def _make_agg():
    mesh = plsc.VectorSubcoreMesh(core_axis_name="c", subcore_axis_name="s",
                                  num_cores=NC, num_subcores=NS)
    return pl.kernel(
        _agg_body,
        out_type=jax.ShapeDtypeStruct((NC, N_P, HH), jnp.float32),
        mesh=mesh,
        compiler_params=pltpu.CompilerParams(use_tc_tiling_on_sc=False),
        scratch_types=[
            pltpu.VMEM((NBUF, K, LANE), jnp.int32),
            pltpu.VMEM((NBUF, K, LANE), jnp.int32),
            pltpu.VMEM((NBUF, K, LANE, HH), jnp.float32),
            pltpu.VMEM_SHARED((ACC_ROWS, HH), jnp.float32),
            pltpu.SemaphoreType.DMA,
            pltpu.SemaphoreType.DMA,
            pltpu.SemaphoreType.DMA,
        ],
    )


BN = 6256
GRID = N_P // BN


def _mlp_math(z, w1, b1, w2, b2):
    a = jnp.maximum(jnp.dot(z, w1, preferred_element_type=jnp.float32) + b1, 0.0)
    return jnp.maximum(jnp.dot(a, w2, preferred_element_type=jnp.float32) + b2, 0.0)


def _mlp_body(z_ref, w1_ref, b1_ref, w2_ref, b2_ref, out_ref):
    z = jnp.concatenate([z_ref[0], z_ref[1]], axis=-1)
    h = _mlp_math(z, w1_ref[...], b1_ref[...], w2_ref[...], b2_ref[...])
    out_ref[0] = h[:, :HH]
    out_ref[1] = h[:, HH:]


def _final_body(z_ref, w1_ref, b1_ref, w2_ref, b2_ref, wf_ref, bf_ref, out_ref):
    z = jnp.concatenate([z_ref[0], z_ref[1]], axis=-1)
    h = _mlp_math(z, w1_ref[...], b1_ref[...], w2_ref[...], b2_ref[...])
    out_ref[...] = jnp.dot(h, wf_ref[...], preferred_element_type=jnp.float32) \
        + bf_ref[...]


_Z_SPEC = pl.BlockSpec((NC, BN, HH), lambda i: (0, i, 0))
_W_SPEC = pl.BlockSpec((H, H), lambda i: (0, 0))
_B_SPEC = pl.BlockSpec((1, H), lambda i: (0, 0))
_P_SPEC = pl.BlockSpec((NC, BN, HH), lambda i: (0, i, 0))
_P_SHAPE = jax.ShapeDtypeStruct((NC, N_P, HH), jnp.float32)

_mlp = pl.pallas_call(
    _mlp_body,
    grid=(GRID,),
    in_specs=[_Z_SPEC, _W_SPEC, _B_SPEC, _W_SPEC, _B_SPEC],
    out_specs=_P_SPEC,
    out_shape=_P_SHAPE,
)

_final = pl.pallas_call(
    _final_body,
    grid=(GRID,),
    in_specs=[_Z_SPEC, _W_SPEC, _B_SPEC, _W_SPEC, _B_SPEC,
              pl.BlockSpec((H, 1), lambda i: (0, 0)),
              pl.BlockSpec((1, 1), lambda i: (0, 0))],
    out_specs=pl.BlockSpec((BN, 1), lambda i: (i, 0)),
    out_shape=jax.ShapeDtypeStruct((N_NODES, 1), jnp.float32),
)


def kernel(x, edge_index, W1_0, b1_0, W2_0, b2_0, W1_1, b1_1, W2_1, b2_1,
           W1_2, b1_2, W2_2, b2_2, Wf, bf):
    lane_ids = jnp.arange(PAD, dtype=jnp.int32) % NS
    src = jnp.concatenate([edge_index[0], lane_ids]).reshape(R_ALLOC, LANE)
    dst = jnp.concatenate([edge_index[1], lane_ids + N_P]).reshape(R_ALLOC, LANE)

    xp = jnp.pad(x, ((0, N_P - N_NODES), (0, 0)))
    hs = jnp.stack([xp[:, :HH], xp[:, HH:]])
    b1s = (b1_0.reshape(1, H), b1_1.reshape(1, H), b1_2.reshape(1, H))
    b2s = (b2_0.reshape(1, H), b2_1.reshape(1, H), b2_2.reshape(1, H))
    W1s = (W1_0, W1_1, W1_2)
    W2s = (W2_0, W2_1, W2_2)
    _agg = _make_agg()

    for i in range(2):
        z = _agg(hs, src, dst)
        hs = _mlp(z, W1s[i], b1s[i], W2s[i], b2s[i])
    z = _agg(hs, src, dst)
    return _final(z, W1s[2], b1s[2], W2s[2], b2s[2], Wf, bf.reshape(1, 1))
